# Initial kernel scaffold; baseline (speedup 1.0000x reference)
import jax
import jax.numpy as jnp
from jax.experimental import pallas as pl


def kernel(x, W_enc, b_enc, W_dec, b_dec):
    raise NotImplementedError("write your pallas kernel here")


# speedup vs baseline: 5.3169x; 5.3169x over previous
import jax
import jax.numpy as jnp
from jax.experimental import pallas as pl
from jax.experimental.pallas import tpu as pltpu

_INPUT_DIM = 768
_LATENT_DIM = 3072
_K = 32
_TM = 256


def _fused_body(x_ref, wenc_ref, benc_ref, wdec_ref, bdec_ref,
                sparse_ref, recon_ref):
    x = x_ref[...]
    latents = jax.lax.dot_general(
        x, wenc_ref[...], (((1,), (0,)), ((), ())),
        preferred_element_type=jnp.float32,
        precision=jax.lax.Precision.DEFAULT) + benc_ref[...]

    iota = jax.lax.broadcasted_iota(jnp.int32, latents.shape, 1)
    neg_inf = jnp.float32(float("-inf"))

    def body(_, carry):
        work, sparse = carry
        m = jnp.max(work, axis=1, keepdims=True)
        eq = work == m
        cand = jnp.where(eq, iota, _LATENT_DIM)
        amin = jnp.min(cand, axis=1, keepdims=True)
        onehot = iota == amin
        work = jnp.where(onehot, neg_inf, work)
        sparse = jnp.where(onehot, latents, sparse)
        return work, sparse

    _, sparse = jax.lax.fori_loop(
        0, _K, body, (latents, jnp.zeros_like(latents)))

    sparse_ref[...] = sparse
    recon_ref[...] = jax.lax.dot_general(
        sparse, wdec_ref[...], (((1,), (0,)), ((), ())),
        preferred_element_type=jnp.float32,
        precision=jax.lax.Precision.DEFAULT) + bdec_ref[...]


def kernel(x, W_enc, b_enc, W_dec, b_dec):
    n = x.shape[0]
    wenc_t = W_enc.T
    wdec_t = W_dec.T
    benc = b_enc.reshape(1, -1)
    bdec = b_dec.reshape(1, -1)

    grid = (n // _TM,)
    sparse, recon = pl.pallas_call(
        _fused_body,
        grid=grid,
        in_specs=[
            pl.BlockSpec((_TM, _INPUT_DIM), lambda i: (i, 0)),
            pl.BlockSpec((_INPUT_DIM, _LATENT_DIM), lambda i: (0, 0)),
            pl.BlockSpec((1, _LATENT_DIM), lambda i: (0, 0)),
            pl.BlockSpec((_LATENT_DIM, _INPUT_DIM), lambda i: (0, 0)),
            pl.BlockSpec((1, _INPUT_DIM), lambda i: (0, 0)),
        ],
        out_specs=[
            pl.BlockSpec((_TM, _LATENT_DIM), lambda i: (i, 0)),
            pl.BlockSpec((_TM, _INPUT_DIM), lambda i: (i, 0)),
        ],
        out_shape=[
            jax.ShapeDtypeStruct((n, _LATENT_DIM), jnp.float32),
            jax.ShapeDtypeStruct((n, _INPUT_DIM), jnp.float32),
        ],
        compiler_params=pltpu.CompilerParams(
            dimension_semantics=("arbitrary",)),
    )(x, wenc_t, benc, wdec_t, bdec)
    return (recon, sparse)


# speedup vs baseline: 13.6890x; 2.5746x over previous
import jax
import jax.numpy as jnp
from jax.experimental import pallas as pl
from jax.experimental.pallas import tpu as pltpu

_INPUT_DIM = 768
_LATENT_DIM = 3072
_K = 32
_TM = 256


def _fused_body(x_ref, wenc_ref, benc_ref, wdec_ref, bdec_ref,
                sparse_ref, recon_ref):
    x = x_ref[...]
    latents = jax.lax.dot_general(
        x, wenc_ref[...], (((1,), (0,)), ((), ())),
        preferred_element_type=jnp.float32,
        precision=jax.lax.Precision.DEFAULT) + benc_ref[...]

    bits = jax.lax.bitcast_convert_type(latents, jnp.int32)
    keys = jnp.where(bits < 0, jnp.int32(-2147483648) - bits, bits)

    lo0 = jnp.full((latents.shape[0], 1), -2139095041, jnp.int32)
    hi0 = jnp.full((latents.shape[0], 1), 2139095041, jnp.int32)

    def bs_body(_, carry):
        lo, hi = carry
        mid = (lo >> 1) + (hi >> 1) + (lo & hi & 1)
        cnt = jnp.sum((keys > mid).astype(jnp.int32), axis=1, keepdims=True)
        ge_k = cnt >= _K
        lo = jnp.where(ge_k, mid + 1, lo)
        hi = jnp.where(ge_k, hi, mid)
        return lo, hi

    lo, _ = jax.lax.fori_loop(0, 32, bs_body, (lo0, hi0))
    vstar = lo

    sel = keys > vstar
    cnt_gt = jnp.sum(sel.astype(jnp.int32), axis=1, keepdims=True)
    deficit = _K - cnt_gt
    eq = keys == vstar
    iota = jax.lax.broadcasted_iota(jnp.int32, latents.shape, 1)

    def tie_body(_, carry):
        sel32, deficit = carry
        pick = jnp.logical_and(eq, sel32 == 0)
        cand = jnp.where(pick, iota, _LATENT_DIM)
        amin = jnp.min(cand, axis=1, keepdims=True)
        add = jnp.logical_and(iota == amin, deficit > 0)
        sel32 = sel32 | add.astype(jnp.int32)
        deficit = deficit - (deficit > 0).astype(jnp.int32)
        return sel32, deficit

    sel32, _ = jax.lax.fori_loop(
        0, 4, tie_body, (sel.astype(jnp.int32), deficit))
    sel = sel32 != 0

    sparse = jnp.where(sel, latents, 0.0)
    sparse_ref[...] = sparse
    recon_ref[...] = jax.lax.dot_general(
        sparse, wdec_ref[...], (((1,), (0,)), ((), ())),
        preferred_element_type=jnp.float32,
        precision=jax.lax.Precision.DEFAULT) + bdec_ref[...]


def kernel(x, W_enc, b_enc, W_dec, b_dec):
    n = x.shape[0]
    wenc_t = W_enc.T
    wdec_t = W_dec.T
    benc = b_enc.reshape(1, -1)
    bdec = b_dec.reshape(1, -1)

    grid = (n // _TM,)
    sparse, recon = pl.pallas_call(
        _fused_body,
        grid=grid,
        in_specs=[
            pl.BlockSpec((_TM, _INPUT_DIM), lambda i: (i, 0)),
            pl.BlockSpec((_INPUT_DIM, _LATENT_DIM), lambda i: (0, 0)),
            pl.BlockSpec((1, _LATENT_DIM), lambda i: (0, 0)),
            pl.BlockSpec((_LATENT_DIM, _INPUT_DIM), lambda i: (0, 0)),
            pl.BlockSpec((1, _INPUT_DIM), lambda i: (0, 0)),
        ],
        out_specs=[
            pl.BlockSpec((_TM, _LATENT_DIM), lambda i: (i, 0)),
            pl.BlockSpec((_TM, _INPUT_DIM), lambda i: (i, 0)),
        ],
        out_shape=[
            jax.ShapeDtypeStruct((n, _LATENT_DIM), jnp.float32),
            jax.ShapeDtypeStruct((n, _INPUT_DIM), jnp.float32),
        ],
        compiler_params=pltpu.CompilerParams(
            dimension_semantics=("arbitrary",)),
    )(x, wenc_t, benc, wdec_t, bdec)
    return (recon, sparse)


# speedup vs baseline: 13.7430x; 1.0039x over previous
import jax
import jax.numpy as jnp
from jax.experimental import pallas as pl
from jax.experimental.pallas import tpu as pltpu

_INPUT_DIM = 768
_LATENT_DIM = 3072
_K = 32
_TM = 256


def _fused_body(x_ref, wenc_ref, benc_ref, wdec_ref, bdec_ref,
                sparse_ref, recon_ref):
    x = x_ref[...]
    latents = jax.lax.dot_general(
        x, wenc_ref[...], (((1,), (0,)), ((), ())),
        preferred_element_type=jnp.float32,
        precision=jax.lax.Precision.DEFAULT) + benc_ref[...]

    bits = jax.lax.bitcast_convert_type(latents, jnp.int32)
    keys = jnp.where(bits < 0, jnp.int32(-2147483648) - bits, bits)

    lo0 = jnp.full((latents.shape[0], 1), -2139095041, jnp.int32)
    hi0 = jnp.full((latents.shape[0], 1), 2139095041, jnp.int32)

    def bs_body(_, carry):
        lo, hi = carry
        mid = (lo >> 1) + (hi >> 1) + (lo & hi & 1)
        cnt = jnp.sum((keys > mid).astype(jnp.int32), axis=1, keepdims=True)
        ge_k = cnt >= _K
        lo = jnp.where(ge_k, mid + 1, lo)
        hi = jnp.where(ge_k, hi, mid)
        return lo, hi

    lo, _ = jax.lax.fori_loop(0, 32, bs_body, (lo0, hi0))
    vstar = lo

    sel = keys > vstar
    cnt_gt = jnp.sum(sel.astype(jnp.int32), axis=1, keepdims=True)
    deficit = _K - cnt_gt
    eq = keys == vstar
    iota = jax.lax.broadcasted_iota(jnp.int32, latents.shape, 1)

    def tie_body(_, carry):
        sel32, deficit = carry
        pick = jnp.logical_and(eq, sel32 == 0)
        cand = jnp.where(pick, iota, _LATENT_DIM)
        amin = jnp.min(cand, axis=1, keepdims=True)
        add = jnp.logical_and(iota == amin, deficit > 0)
        sel32 = sel32 | add.astype(jnp.int32)
        deficit = deficit - (deficit > 0).astype(jnp.int32)
        return sel32, deficit

    sel32, _ = jax.lax.fori_loop(
        0, 4, tie_body, (sel.astype(jnp.int32), deficit))
    sel = sel32 != 0

    sparse = jnp.where(sel, latents, 0.0)
    sparse_ref[...] = sparse
    recon_ref[...] = jax.lax.dot_general(
        sparse.astype(jnp.bfloat16), wdec_ref[...], (((1,), (0,)), ((), ())),
        preferred_element_type=jnp.float32,
        precision=jax.lax.Precision.DEFAULT) + bdec_ref[...]


def kernel(x, W_enc, b_enc, W_dec, b_dec):
    n = x.shape[0]
    wenc_t = W_enc.T
    wdec_t = W_dec.T.astype(jnp.bfloat16)
    benc = b_enc.reshape(1, -1)
    bdec = b_dec.reshape(1, -1)

    grid = (n // _TM,)
    sparse, recon = pl.pallas_call(
        _fused_body,
        grid=grid,
        in_specs=[
            pl.BlockSpec((_TM, _INPUT_DIM), lambda i: (i, 0)),
            pl.BlockSpec((_INPUT_DIM, _LATENT_DIM), lambda i: (0, 0)),
            pl.BlockSpec((1, _LATENT_DIM), lambda i: (0, 0)),
            pl.BlockSpec((_LATENT_DIM, _INPUT_DIM), lambda i: (0, 0)),
            pl.BlockSpec((1, _INPUT_DIM), lambda i: (0, 0)),
        ],
        out_specs=[
            pl.BlockSpec((_TM, _LATENT_DIM), lambda i: (i, 0)),
            pl.BlockSpec((_TM, _INPUT_DIM), lambda i: (i, 0)),
        ],
        out_shape=[
            jax.ShapeDtypeStruct((n, _LATENT_DIM), jnp.float32),
            jax.ShapeDtypeStruct((n, _INPUT_DIM), jnp.float32),
        ],
        compiler_params=pltpu.CompilerParams(
            dimension_semantics=("arbitrary",)),
    )(x, wenc_t, benc, wdec_t, bdec)
    return (recon, sparse)
